# baseline (speedup 1.0000x reference)
import jax
import jax.numpy as jnp
from jax.experimental import pallas as pl
from jax.experimental.pallas import tpu as pltpu

_EPS = 1e-5


def _mean_inv(s, css, gamma, n_rows):
    mean = s / n_rows
    var = css / n_rows
    inv = gamma / jnp.sqrt(var + _EPS)
    return mean, inv


def _layer_a(adj, h, w1, b1, *, block_rows):
    n = adj.shape[0]
    d = h.shape[1]
    hd = w1.shape[1]
    nb = n // block_rows

    def kern(adj_ref, h_ref, w1_ref, b1_ref, h1_ref, s_ref):
        i = pl.program_id(0)
        pooled = jnp.dot(adj_ref[...].astype(jnp.bfloat16),
                         h_ref[...].astype(jnp.bfloat16),
                         preferred_element_type=jnp.float32)
        pooled = pooled.astype(jnp.bfloat16).astype(jnp.float32)
        h1 = jnp.dot(pooled, w1_ref[...],
                     preferred_element_type=jnp.float32) + b1_ref[...]
        h1_ref[...] = h1

        @pl.when(i == 0)
        def _():
            s_ref[...] = jnp.zeros_like(s_ref)

        s_ref[...] += jnp.sum(h1, axis=0, keepdims=True)

    return pl.pallas_call(
        kern,
        grid=(nb,),
        in_specs=[
            pl.BlockSpec((block_rows, n), lambda i: (i, 0)),
            pl.BlockSpec((n, d), lambda i: (0, 0)),
            pl.BlockSpec((d, hd), lambda i: (0, 0)),
            pl.BlockSpec((1, hd), lambda i: (0, 0)),
        ],
        out_specs=[
            pl.BlockSpec((block_rows, hd), lambda i: (i, 0)),
            pl.BlockSpec((1, hd), lambda i: (0, 0)),
        ],
        out_shape=[
            jax.ShapeDtypeStruct((n, hd), jnp.float32),
            jax.ShapeDtypeStruct((1, hd), jnp.float32),
        ],
        compiler_params=pltpu.CompilerParams(
            dimension_semantics=("arbitrary",)),
    )(adj, h, w1, b1.reshape(1, -1))


def _col_css(hx, s, *, block_rows):
    n, hd = hx.shape
    nb = n // block_rows

    def kern(h_ref, s_ref, css_ref):
        i = pl.program_id(0)

        @pl.when(i == 0)
        def _():
            css_ref[...] = jnp.zeros_like(css_ref)

        c = h_ref[...] - s_ref[...] / n
        css_ref[...] += jnp.sum(c * c, axis=0, keepdims=True)

    return pl.pallas_call(
        kern,
        grid=(nb,),
        in_specs=[
            pl.BlockSpec((block_rows, hd), lambda i: (i, 0)),
            pl.BlockSpec((1, hd), lambda i: (0, 0)),
        ],
        out_specs=pl.BlockSpec((1, hd), lambda i: (0, 0)),
        out_shape=jax.ShapeDtypeStruct((1, hd), jnp.float32),
        compiler_params=pltpu.CompilerParams(
            dimension_semantics=("arbitrary",)),
    )(hx, s)


def _layer_b(h1, s1, css1, g1, be1, w2, b2, *, block_rows):
    n, hd = h1.shape
    nb = n // block_rows

    def kern(h1_ref, s1_ref, css1_ref, g1_ref, be1_ref, w2_ref, b2_ref,
             h2_ref, s_ref):
        i = pl.program_id(0)
        mean, inv = _mean_inv(s1_ref[...], css1_ref[...], g1_ref[...], n)
        h1n = jnp.maximum((h1_ref[...] - mean) * inv + be1_ref[...], 0.0)
        h2 = jnp.dot(h1n, w2_ref[...],
                     preferred_element_type=jnp.float32) + b2_ref[...]
        h2_ref[...] = h2

        @pl.when(i == 0)
        def _():
            s_ref[...] = jnp.zeros_like(s_ref)

        s_ref[...] += jnp.sum(h2, axis=0, keepdims=True)

    return pl.pallas_call(
        kern,
        grid=(nb,),
        in_specs=[
            pl.BlockSpec((block_rows, hd), lambda i: (i, 0)),
            pl.BlockSpec((1, hd), lambda i: (0, 0)),
            pl.BlockSpec((1, hd), lambda i: (0, 0)),
            pl.BlockSpec((1, hd), lambda i: (0, 0)),
            pl.BlockSpec((1, hd), lambda i: (0, 0)),
            pl.BlockSpec((hd, hd), lambda i: (0, 0)),
            pl.BlockSpec((1, hd), lambda i: (0, 0)),
        ],
        out_specs=[
            pl.BlockSpec((block_rows, hd), lambda i: (i, 0)),
            pl.BlockSpec((1, hd), lambda i: (0, 0)),
        ],
        out_shape=[
            jax.ShapeDtypeStruct((n, hd), jnp.float32),
            jax.ShapeDtypeStruct((1, hd), jnp.float32),
        ],
        compiler_params=pltpu.CompilerParams(
            dimension_semantics=("arbitrary",)),
    )(h1, s1, css1, g1.reshape(1, -1), be1.reshape(1, -1), w2,
      b2.reshape(1, -1))


def _bn_relu(h2, s2, css2, g, b, *, block_rows):
    n, hd = h2.shape
    nb = n // block_rows

    def kern(h2_ref, s_ref, css_ref, g_ref, b_ref, out_ref):
        mean, inv = _mean_inv(s_ref[...], css_ref[...], g_ref[...], n)
        out_ref[...] = jnp.maximum((h2_ref[...] - mean) * inv + b_ref[...],
                                   0.0)

    return pl.pallas_call(
        kern,
        grid=(nb,),
        in_specs=[
            pl.BlockSpec((block_rows, hd), lambda i: (i, 0)),
            pl.BlockSpec((1, hd), lambda i: (0, 0)),
            pl.BlockSpec((1, hd), lambda i: (0, 0)),
            pl.BlockSpec((1, hd), lambda i: (0, 0)),
            pl.BlockSpec((1, hd), lambda i: (0, 0)),
        ],
        out_specs=pl.BlockSpec((block_rows, hd), lambda i: (i, 0)),
        out_shape=jax.ShapeDtypeStruct((n, hd), jnp.float32),
        compiler_params=pltpu.CompilerParams(
            dimension_semantics=("arbitrary",)),
    )(h2, s2, css2, g.reshape(1, -1), b.reshape(1, -1))


def _pool_tail(h2, s2, css2, g, b, graph_pool):
    n, hd = h2.shape
    ng = graph_pool.shape[0]

    def kern(h2_ref, s_ref, css_ref, g_ref, b_ref, gp_ref,
             pooled_ref, hn_ref):
        mean, inv = _mean_inv(s_ref[...], css_ref[...], g_ref[...], n)
        h = jnp.maximum((h2_ref[...] - mean) * inv + b_ref[...], 0.0)
        hn_ref[...] = h
        pooled_ref[...] = jnp.dot(gp_ref[...], h,
                                  preferred_element_type=jnp.float32)

    return pl.pallas_call(
        kern,
        in_specs=[
            pl.BlockSpec((n, hd), lambda: (0, 0)),
            pl.BlockSpec((1, hd), lambda: (0, 0)),
            pl.BlockSpec((1, hd), lambda: (0, 0)),
            pl.BlockSpec((1, hd), lambda: (0, 0)),
            pl.BlockSpec((1, hd), lambda: (0, 0)),
            pl.BlockSpec((ng, n), lambda: (0, 0)),
        ],
        out_specs=[
            pl.BlockSpec((ng, hd), lambda: (0, 0)),
            pl.BlockSpec((n, hd), lambda: (0, 0)),
        ],
        out_shape=[
            jax.ShapeDtypeStruct((ng, hd), jnp.float32),
            jax.ShapeDtypeStruct((n, hd), jnp.float32),
        ],
    )(h2, s2, css2, g.reshape(1, -1), b.reshape(1, -1), graph_pool)


def kernel(x, graph_pool, adj, params):
    h = x
    num_layers = len(params)
    h2 = s2 = css2 = None
    for li, p in enumerate(params):
        h1, s1 = _layer_a(adj, h, p['W1'], p['b1'], block_rows=400)
        css1 = _col_css(h1, s1, block_rows=2000)
        h2, s2 = _layer_b(h1, s1, css1, p['g1'], p['be1'],
                          p['W2'], p['b2'], block_rows=2000)
        css2 = _col_css(h2, s2, block_rows=2000)
        if li < num_layers - 1:
            h = _bn_relu(h2, s2, css2, p['bn_g'], p['bn_b'],
                         block_rows=2000)
    pooled_h, h_nodes = _pool_tail(h2, s2, css2, params[-1]['bn_g'],
                                   params[-1]['bn_b'], graph_pool)
    return (pooled_h, h_nodes)
